# baseline (speedup 1.0000x reference)
import functools

import jax
import jax.numpy as jnp
from jax import lax
from jax.experimental import pallas as pl
from jax.experimental.pallas import tpu as pltpu
from jax.experimental.pallas import tpu_sc as plsc

_N = 10000
_E = 320000
_NC = 2
_NS = 16
_NW = _NC * _NS
_C = 128
_NCH = -(-_E // (_NW * _C))
_EPAD = _NW * _NCH * _C
_RPT = 640
_NPAD = _RPT * _NS
_PAD_NODE = _N + 16

_mesh = plsc.VectorSubcoreMesh(core_axis_name="c", subcore_axis_name="s")


def _seg_sum(feat, idx3, d, with_cnt):
  nbuf = 2 if d == 128 else 4
  depth = 1 if nbuf == 2 else 2

  def body(*refs):
    if with_cnt:
      (feat_h, idx_h, zf_h, ones_h, zc_h, agg_o, cnt_o) = refs[:7]
      refs = refs[7:]
    else:
      (feat_h, idx_h, zf_h, agg_o) = refs[:4]
      refs = refs[4:]
    idx2 = list(refs[:4])
    rows = list(refs[4:4 + nbuf])
    refs = refs[4 + nbuf:]
    if with_cnt:
      acc, ones_v, cacc = refs[:3]
      refs = refs[3:]
    else:
      acc = refs[0]
      refs = refs[1:]
    isem = list(refs[:4])
    gsem = list(refs[4:4 + nbuf])
    ssem = list(refs[4 + nbuf:4 + 2 * nbuf])
    if with_cnt:
      csem = list(refs[4 + 2 * nbuf:4 + 3 * nbuf])

    c = lax.axis_index("c")
    s = lax.axis_index("s")
    wid = c * _NS + s
    r0 = s * _RPT

    def idx_load(j, q):
      return pltpu.async_copy(idx_h.at[wid, j], idx2[q], isem[q])

    def gather(j, q, b):
      return pltpu.async_copy(feat_h.at[idx2[q].at[0]], rows[b], gsem[b])

    def scatter(b, q):
      pltpu.async_copy(rows[b], acc.at[idx2[q].at[1]], ssem[b], add=True)
      if with_cnt:
        pltpu.async_copy(ones_v, cacc.at[idx2[q].at[1]], csem[b], add=True)

    def wait_scatter(b, q):
      pltpu.make_async_copy(rows[b], acc.at[idx2[q].at[1]], ssem[b]).wait()
      if with_cnt:
        pltpu.make_async_copy(ones_v, cacc.at[idx2[q].at[1]], csem[b]).wait()

    pltpu.sync_copy(zf_h.at[pl.ds(r0, _RPT)], acc.at[pl.ds(r0, _RPT)])
    if with_cnt:
      pltpu.sync_copy(zc_h.at[pl.ds(r0, _RPT)], cacc.at[pl.ds(r0, _RPT)])
      pltpu.sync_copy(ones_h, ones_v)
    descs = [idx_load(t, t) for t in range(3)]
    plsc.subcore_barrier()
    for t in range(depth):
      descs[t].wait()
      gather(t, t, t % nbuf)

    def iter4(jj, carry):
      for q in range(4):
        j = jj * 4 + q
        b = q % nbuf

        @pl.when(j < _NCH)
        def _():
          pltpu.make_async_copy(feat_h.at[idx2[q].at[0]], rows[b],
                                gsem[b]).wait()
          scatter(b, q)

          @pl.when(j >= 1)
          def _():
            wait_scatter((q + 3) % 4 % nbuf, (q + 3) % 4)

          @pl.when(j + depth < _NCH)
          def _():
            pltpu.make_async_copy(idx_h.at[wid, 0], idx2[(q + depth) % 4],
                                  isem[(q + depth) % 4]).wait()
            gather(j + depth, (q + depth) % 4, (q + depth) % 4 % nbuf)

          @pl.when(j + 3 < _NCH)
          def _():
            idx_load(j + 3, (q + 3) % 4)
      return carry

    lax.fori_loop(0, (_NCH + 3) // 4, iter4, 0)
    wait_scatter((_NCH - 1) % 4 % nbuf, (_NCH - 1) % 4)
    plsc.subcore_barrier()
    pltpu.sync_copy(acc.at[pl.ds(r0, _RPT)], agg_o.at[c, pl.ds(r0, _RPT)])
    if with_cnt:
      pltpu.sync_copy(cacc.at[pl.ds(r0, _RPT)], cnt_o.at[c, pl.ds(r0, _RPT)])

  outs = [jax.ShapeDtypeStruct((_NC, _NPAD, d), jnp.float32)]
  inputs = [feat, idx3, jnp.zeros((_NPAD, d), jnp.float32)]
  scratch = (
      [pltpu.VMEM((2, _C), jnp.int32) for _ in range(4)]
      + [pltpu.VMEM((_C, d), jnp.float32) for _ in range(nbuf)]
  )
  if with_cnt:
    outs.append(jax.ShapeDtypeStruct((_NC, _NPAD, 8), jnp.float32))
    inputs += [jnp.ones((_C, 8), jnp.float32),
               jnp.zeros((_NPAD, 8), jnp.float32)]
    scratch += [pltpu.VMEM_SHARED((_NPAD, d), jnp.float32),
                pltpu.VMEM((_C, 8), jnp.float32),
                pltpu.VMEM_SHARED((_NPAD, 8), jnp.float32)]
  else:
    scratch += [pltpu.VMEM_SHARED((_NPAD, d), jnp.float32)]
  nsem = (4 + 3 * nbuf) if with_cnt else (4 + 2 * nbuf)
  scratch += [pltpu.SemaphoreType.DMA for _ in range(nsem)]

  f = pl.kernel(
      body,
      out_type=tuple(outs),
      mesh=_mesh,
      compiler_params=pltpu.CompilerParams(use_tc_tiling_on_sc=False),
      scratch_types=tuple(scratch),
  )
  return f(*inputs)


def _dot_t(a, w):
  return lax.dot_general(a, w, (((1,), (1,)), ((), ())),
                         precision=lax.Precision.HIGHEST,
                         preferred_element_type=jnp.float32)


def _mean(agg_r, cnt_r):
  cnt = cnt_r[0, :, 0:1] + cnt_r[1, :, 0:1]
  inv = 1.0 / jnp.maximum(cnt, 1.0)
  return (agg_r[0] + agg_r[1]) * inv


def _elu(h):
  return jnp.where(h > 0, h, jnp.exp(jnp.minimum(h, 0.0)) - 1.0)


_BM = 1000


def _t1_body(x_r, a_r, c_r, w1l_r, b1_r, w1r_r, w2l_r, b2_r, w2r_r,
             p2_o, r2_o):
  mean = _mean(a_r, c_r)
  h1 = _dot_t(mean, w1l_r[...]) + b1_r[...] + _dot_t(x_r[...], w1r_r[...])
  h1 = _elu(h1)
  p2_o[...] = _dot_t(h1, w2l_r[...])
  r2_o[...] = _dot_t(h1, w2r_r[...]) + b2_r[...]


def _t1(x, agg1, cnt, w1l, b1, w1r, w2l, b2, w2r):
  grid = (_N // _BM,)
  full = lambda shape: pl.BlockSpec(shape, lambda i: (0,) * len(shape))
  return pl.pallas_call(
      _t1_body,
      grid=grid,
      in_specs=[
          pl.BlockSpec((_BM, 128), lambda i: (i, 0)),
          pl.BlockSpec((_NC, _BM, 128), lambda i: (0, i, 0)),
          pl.BlockSpec((_NC, _BM, 8), lambda i: (0, i, 0)),
          full((256, 128)), full((1, 256)), full((256, 128)),
          full((32, 256)), full((1, 32)), full((32, 256)),
      ],
      out_specs=[
          pl.BlockSpec((_BM, 32), lambda i: (i, 0)),
          pl.BlockSpec((_BM, 32), lambda i: (i, 0)),
      ],
      out_shape=[
          jax.ShapeDtypeStruct((_N, 32), jnp.float32),
          jax.ShapeDtypeStruct((_N, 32), jnp.float32),
      ],
  )(x, agg1, cnt, w1l, b1, w1r, w2l, b2, w2r)


def _t2_body(a_r, c_r, r2_r, h2_o):
  h2_o[...] = _elu(_mean(a_r, c_r) + r2_r[...])


def _t2(agg2, cnt, r2):
  grid = (_N // _BM,)
  return pl.pallas_call(
      _t2_body,
      grid=grid,
      in_specs=[
          pl.BlockSpec((_NC, _BM, 32), lambda i: (0, i, 0)),
          pl.BlockSpec((_NC, _BM, 8), lambda i: (0, i, 0)),
          pl.BlockSpec((_BM, 32), lambda i: (i, 0)),
      ],
      out_specs=pl.BlockSpec((_BM, 32), lambda i: (i, 0)),
      out_shape=jax.ShapeDtypeStruct((_N, 32), jnp.float32),
  )(agg2, cnt, r2)


def _t3_body(a_r, c_r, h2_r, w3l_r, b3_r, w3r_r, out_o):
  mean = _mean(a_r, c_r)
  out_o[...] = (_dot_t(mean, w3l_r[...]) + b3_r[...]
                + _dot_t(h2_r[...], w3r_r[...]))


def _t3(agg3, cnt, h2, w3l, b3, w3r):
  grid = (_N // _BM,)
  full = lambda shape: pl.BlockSpec(shape, lambda i: (0,) * len(shape))
  return pl.pallas_call(
      _t3_body,
      grid=grid,
      in_specs=[
          pl.BlockSpec((_NC, _BM, 32), lambda i: (0, i, 0)),
          pl.BlockSpec((_NC, _BM, 8), lambda i: (0, i, 0)),
          pl.BlockSpec((_BM, 32), lambda i: (i, 0)),
          full((64, 32)), full((1, 64)), full((64, 32)),
      ],
      out_specs=pl.BlockSpec((_BM, 64), lambda i: (i, 0)),
      out_shape=jax.ShapeDtypeStruct((_N, 64), jnp.float32),
  )(agg3, cnt, h2, w3l, b3, w3r)


def kernel(x, edge_index, W1l, b1, W1r, W2l, b2, W2r, W3l, b3, W3r):
  ei = edge_index.astype(jnp.int32)
  src = jnp.pad(ei[0], (0, _EPAD - _E))
  dst = jnp.pad(ei[1], (0, _EPAD - _E), constant_values=_PAD_NODE)
  idx3 = jnp.stack([src.reshape(_NW, _NCH, _C),
                    dst.reshape(_NW, _NCH, _C)], axis=2)

  agg1, cnt = _seg_sum(x, idx3, 128, True)
  p2, r2 = _t1(x, agg1, cnt, W1l, b1.reshape(1, 256), W1r,
               W2l, b2.reshape(1, 32), W2r)
  agg2 = _seg_sum(p2, idx3, 32, False)[0]
  h2 = _t2(agg2, cnt, r2)
  agg3 = _seg_sum(h2, idx3, 32, False)[0]
  return _t3(agg3, cnt, h2, W3l, b3.reshape(1, 64), W3r)


# speedup vs baseline: 1.0287x; 1.0287x over previous
import functools

import jax
import jax.numpy as jnp
from jax import lax
from jax.experimental import pallas as pl
from jax.experimental.pallas import tpu as pltpu
from jax.experimental.pallas import tpu_sc as plsc

_N = 10000
_E = 320000
_NC = 2
_NS = 16
_NW = _NC * _NS
_C = 128
_NCH = -(-_E // (_NW * _C))
_EPAD = _NW * _NCH * _C
_RPT = 640
_NPAD = _RPT * _NS
_PAD_NODE = _N + 16

_mesh = plsc.VectorSubcoreMesh(core_axis_name="c", subcore_axis_name="s")


def _seg_sum(feat, idx3, d, with_cnt):
  nbuf = 2 if d == 128 else 4
  depth = 1 if nbuf == 2 else 2

  def body(*refs):
    if with_cnt:
      (feat_h, idx_h, zf_h, ones_h, zc_h, agg_o, cnt_o) = refs[:7]
      refs = refs[7:]
    else:
      (feat_h, idx_h, zf_h, agg_o) = refs[:4]
      refs = refs[4:]
    idx2 = list(refs[:4])
    rows = list(refs[4:4 + nbuf])
    refs = refs[4 + nbuf:]
    if with_cnt:
      acc, ones_v, cacc = refs[:3]
      refs = refs[3:]
    else:
      acc = refs[0]
      refs = refs[1:]
    isem = list(refs[:4])
    gsem = list(refs[4:4 + nbuf])
    ssem = list(refs[4 + nbuf:4 + 2 * nbuf])
    if with_cnt:
      csem = list(refs[4 + 2 * nbuf:4 + 3 * nbuf])

    c = lax.axis_index("c")
    s = lax.axis_index("s")
    wid = c * _NS + s
    r0 = s * _RPT

    def idx_load(j, q):
      return pltpu.async_copy(idx_h.at[wid, j], idx2[q], isem[q])

    def gather(j, q, b):
      return pltpu.async_copy(feat_h.at[idx2[q].at[0]], rows[b], gsem[b])

    def scatter(b, q):
      pltpu.async_copy(rows[b], acc.at[idx2[q].at[1]], ssem[b], add=True)
      if with_cnt:
        pltpu.async_copy(ones_v, cacc.at[idx2[q].at[1]], csem[b], add=True)

    def wait_scatter(b, q):
      pltpu.make_async_copy(rows[b], acc.at[idx2[q].at[1]], ssem[b]).wait()
      if with_cnt:
        pltpu.make_async_copy(ones_v, cacc.at[idx2[q].at[1]], csem[b]).wait()

    pltpu.sync_copy(zf_h.at[pl.ds(r0, _RPT)], acc.at[pl.ds(r0, _RPT)])
    if with_cnt:
      pltpu.sync_copy(zc_h.at[pl.ds(r0, _RPT)], cacc.at[pl.ds(r0, _RPT)])
      pltpu.sync_copy(ones_h, ones_v)
    descs = [idx_load(t, t) for t in range(3)]
    plsc.subcore_barrier()
    for t in range(depth):
      descs[t].wait()
      gather(t, t, t % nbuf)

    def iter4(jj, carry):
      for q in range(4):
        j = jj * 4 + q
        b = q % nbuf

        @pl.when(j < _NCH)
        def _():
          pltpu.make_async_copy(feat_h.at[idx2[q].at[0]], rows[b],
                                gsem[b]).wait()
          scatter(b, q)

          @pl.when(j >= 1)
          def _():
            wait_scatter((q + 3) % 4 % nbuf, (q + 3) % 4)

          @pl.when(j + depth < _NCH)
          def _():
            pltpu.make_async_copy(idx_h.at[wid, 0], idx2[(q + depth) % 4],
                                  isem[(q + depth) % 4]).wait()
            gather(j + depth, (q + depth) % 4, (q + depth) % 4 % nbuf)

          @pl.when(j + 3 < _NCH)
          def _():
            idx_load(j + 3, (q + 3) % 4)
      return carry

    lax.fori_loop(0, (_NCH + 3) // 4, iter4, 0)
    wait_scatter((_NCH - 1) % 4 % nbuf, (_NCH - 1) % 4)
    plsc.subcore_barrier()
    pltpu.sync_copy(acc.at[pl.ds(r0, _RPT)], agg_o.at[c, pl.ds(r0, _RPT)])
    if with_cnt:
      pltpu.sync_copy(cacc.at[pl.ds(r0, _RPT)], cnt_o.at[c, pl.ds(r0, _RPT)])

  outs = [jax.ShapeDtypeStruct((_NC, _NPAD, d), jnp.float32)]
  inputs = [feat, idx3, jnp.zeros((_NPAD, d), jnp.float32)]
  scratch = (
      [pltpu.VMEM((2, _C), jnp.int32) for _ in range(4)]
      + [pltpu.VMEM((_C, d), jnp.float32) for _ in range(nbuf)]
  )
  if with_cnt:
    outs.append(jax.ShapeDtypeStruct((_NC, _NPAD, 8), jnp.float32))
    inputs += [jnp.ones((_C, 8), jnp.float32),
               jnp.zeros((_NPAD, 8), jnp.float32)]
    scratch += [pltpu.VMEM_SHARED((_NPAD, d), jnp.float32),
                pltpu.VMEM((_C, 8), jnp.float32),
                pltpu.VMEM_SHARED((_NPAD, 8), jnp.float32)]
  else:
    scratch += [pltpu.VMEM_SHARED((_NPAD, d), jnp.float32)]
  nsem = (4 + 3 * nbuf) if with_cnt else (4 + 2 * nbuf)
  scratch += [pltpu.SemaphoreType.DMA for _ in range(nsem)]

  f = pl.kernel(
      body,
      out_type=tuple(outs),
      mesh=_mesh,
      compiler_params=pltpu.CompilerParams(use_tc_tiling_on_sc=False),
      scratch_types=tuple(scratch),
  )
  return f(*inputs)


def _dot_t(a, w):
  return lax.dot_general(a, w, (((1,), (1,)), ((), ())),
                         precision=lax.Precision.HIGHEST,
                         preferred_element_type=jnp.float32)


def _mean(agg_r, cnt_r):
  cnt = cnt_r[0, :, 0:1] + cnt_r[1, :, 0:1]
  inv = 1.0 / jnp.maximum(cnt, 1.0)
  return (agg_r[0] + agg_r[1]) * inv


def _elu(h):
  return jnp.where(h > 0, h, jnp.exp(jnp.minimum(h, 0.0)) - 1.0)


_BM = 1000


def _t1_body(x_r, a_r, c_r, w1l_r, b1_r, w1r_r, w2l_r, b2_r, w2r_r,
             p2_o, r2_o):
  mean = _mean(a_r, c_r)
  h1 = _dot_t(mean, w1l_r[...]) + b1_r[...] + _dot_t(x_r[...], w1r_r[...])
  h1 = _elu(h1)
  p2_o[...] = _dot_t(h1, w2l_r[...])
  r2_o[...] = _dot_t(h1, w2r_r[...]) + b2_r[...]


def _t1(x, agg1, cnt, w1l, b1, w1r, w2l, b2, w2r):
  grid = (_N // _BM,)
  full = lambda shape: pl.BlockSpec(shape, lambda i: (0,) * len(shape))
  return pl.pallas_call(
      _t1_body,
      grid=grid,
      in_specs=[
          pl.BlockSpec((_BM, 128), lambda i: (i, 0)),
          pl.BlockSpec((_NC, _BM, 128), lambda i: (0, i, 0)),
          pl.BlockSpec((_NC, _BM, 8), lambda i: (0, i, 0)),
          full((256, 128)), full((1, 256)), full((256, 128)),
          full((32, 256)), full((1, 32)), full((32, 256)),
      ],
      out_specs=[
          pl.BlockSpec((_BM, 32), lambda i: (i, 0)),
          pl.BlockSpec((_BM, 32), lambda i: (i, 0)),
      ],
      out_shape=[
          jax.ShapeDtypeStruct((_N, 32), jnp.float32),
          jax.ShapeDtypeStruct((_N, 32), jnp.float32),
      ],
  )(x, agg1, cnt, w1l, b1, w1r, w2l, b2, w2r)


def _t2_body(a_r, c_r, r2_r, h2_o):
  h2_o[...] = _elu(_mean(a_r, c_r) + r2_r[...])


def _t2(agg2, cnt, r2):
  grid = (_N // _BM,)
  return pl.pallas_call(
      _t2_body,
      grid=grid,
      in_specs=[
          pl.BlockSpec((_NC, _BM, 32), lambda i: (0, i, 0)),
          pl.BlockSpec((_NC, _BM, 8), lambda i: (0, i, 0)),
          pl.BlockSpec((_BM, 32), lambda i: (i, 0)),
      ],
      out_specs=pl.BlockSpec((_BM, 32), lambda i: (i, 0)),
      out_shape=jax.ShapeDtypeStruct((_N, 32), jnp.float32),
  )(agg2, cnt, r2)


def _t3_body(a_r, c_r, h2_r, w3l_r, b3_r, w3r_r, out_o):
  mean = _mean(a_r, c_r)
  out_o[...] = (_dot_t(mean, w3l_r[...]) + b3_r[...]
                + _dot_t(h2_r[...], w3r_r[...]))


def _t3(agg3, cnt, h2, w3l, b3, w3r):
  grid = (_N // _BM,)
  full = lambda shape: pl.BlockSpec(shape, lambda i: (0,) * len(shape))
  return pl.pallas_call(
      _t3_body,
      grid=grid,
      in_specs=[
          pl.BlockSpec((_NC, _BM, 32), lambda i: (0, i, 0)),
          pl.BlockSpec((_NC, _BM, 8), lambda i: (0, i, 0)),
          pl.BlockSpec((_BM, 32), lambda i: (i, 0)),
          full((64, 32)), full((1, 64)), full((64, 32)),
      ],
      out_specs=pl.BlockSpec((_BM, 64), lambda i: (i, 0)),
      out_shape=jax.ShapeDtypeStruct((_N, 64), jnp.float32),
  )(agg3, cnt, h2, w3l, b3, w3r)


def kernel(x, edge_index, W1l, b1, W1r, W2l, b2, W2r, W3l, b3, W3r):
  ei = edge_index.astype(jnp.int32)
  src = jnp.pad(ei[0], (0, _EPAD - _E))
  dst = jnp.pad(ei[1], (0, _EPAD - _E), constant_values=_PAD_NODE)
  idx3 = jnp.stack([src.reshape(_NW, _NCH, _C),
                    dst.reshape(_NW, _NCH, _C)], axis=2)

  xp = jnp.pad(x, ((0, _NPAD - _N), (0, 0)))
  agg1, cnt = _seg_sum(xp, idx3, 128, True)
  p2, r2 = _t1(x, agg1, cnt, W1l, b1.reshape(1, 256), W1r,
               W2l, b2.reshape(1, 32), W2r)
  agg2 = _seg_sum(p2, idx3, 32, False)[0]
  h2 = _t2(agg2, cnt, r2)
  agg3 = _seg_sum(h2, idx3, 32, False)[0]
  return _t3(agg3, cnt, h2, W3l, b3.reshape(1, 64), W3r)


# speedup vs baseline: 1.1289x; 1.0974x over previous
import functools

import jax
import jax.numpy as jnp
from jax import lax
from jax.experimental import pallas as pl
from jax.experimental.pallas import tpu as pltpu
from jax.experimental.pallas import tpu_sc as plsc

_N = 10000
_E = 320000
_NC = 2
_NS = 16
_NW = _NC * _NS
_C = 128
_NCH = -(-_E // (_NW * _C))
_EPAD = _NW * _NCH * _C
_RPT = 640
_NPAD = _RPT * _NS
_PAD_NODE = _N + 16

_mesh = plsc.VectorSubcoreMesh(core_axis_name="c", subcore_axis_name="s")


def _seg_sum(feat, idx3, d, with_cnt):
  nbuf = 2 if d == 128 else 4
  depth = 1 if nbuf == 2 else 2
  stage = d <= 32
  srows = _N // _NS

  def body(*refs):
    if with_cnt:
      (feat_h, idx_h, zf_h, ones_h, zc_h, agg_o, cnt_o) = refs[:7]
      refs = refs[7:]
    else:
      (feat_h, idx_h, zf_h, agg_o) = refs[:4]
      refs = refs[4:]
    idx2 = list(refs[:4])
    rows = list(refs[4:4 + nbuf])
    refs = refs[4 + nbuf:]
    if stage:
      feat_s = refs[0]
      refs = refs[1:]
    else:
      feat_s = None
    if with_cnt:
      acc, ones_v, cacc = refs[:3]
      refs = refs[3:]
    else:
      acc = refs[0]
      refs = refs[1:]
    isem = list(refs[:4])
    gsem = list(refs[4:4 + nbuf])
    ssem = list(refs[4 + nbuf:4 + 2 * nbuf])
    if with_cnt:
      csem = list(refs[4 + 2 * nbuf:4 + 3 * nbuf])

    c = lax.axis_index("c")
    s = lax.axis_index("s")
    wid = c * _NS + s
    r0 = s * _RPT

    def idx_load(j, q):
      return pltpu.async_copy(idx_h.at[wid, j], idx2[q], isem[q])

    feat_src = feat_s if stage else feat_h

    def gather(j, q, b):
      return pltpu.async_copy(feat_src.at[idx2[q].at[0]], rows[b], gsem[b])

    def scatter(b, q):
      pltpu.async_copy(rows[b], acc.at[idx2[q].at[1]], ssem[b], add=True)
      if with_cnt:
        pltpu.async_copy(ones_v, cacc.at[idx2[q].at[1]], csem[b], add=True)

    def wait_scatter(b, q):
      pltpu.make_async_copy(rows[b], acc.at[idx2[q].at[1]], ssem[b]).wait()
      if with_cnt:
        pltpu.make_async_copy(ones_v, cacc.at[idx2[q].at[1]], csem[b]).wait()

    pltpu.sync_copy(zf_h.at[pl.ds(r0, _RPT)], acc.at[pl.ds(r0, _RPT)])
    if stage:
      pltpu.sync_copy(feat_h.at[pl.ds(s * srows, srows)],
                      feat_s.at[pl.ds(s * srows, srows)])
    if with_cnt:
      pltpu.sync_copy(zc_h.at[pl.ds(r0, _RPT)], cacc.at[pl.ds(r0, _RPT)])
      pltpu.sync_copy(ones_h, ones_v)
    descs = [idx_load(t, t) for t in range(3)]
    plsc.subcore_barrier()
    for t in range(depth):
      descs[t].wait()
      gather(t, t, t % nbuf)

    def iter4(jj, carry):
      for q in range(4):
        j = jj * 4 + q
        b = q % nbuf

        @pl.when(j < _NCH)
        def _():
          pltpu.make_async_copy(feat_src.at[idx2[q].at[0]], rows[b],
                                gsem[b]).wait()
          scatter(b, q)

          @pl.when(j >= 1)
          def _():
            wait_scatter((q + 3) % 4 % nbuf, (q + 3) % 4)

          @pl.when(j + depth < _NCH)
          def _():
            pltpu.make_async_copy(idx_h.at[wid, 0], idx2[(q + depth) % 4],
                                  isem[(q + depth) % 4]).wait()
            gather(j + depth, (q + depth) % 4, (q + depth) % 4 % nbuf)

          @pl.when(j + 3 < _NCH)
          def _():
            idx_load(j + 3, (q + 3) % 4)
      return carry

    lax.fori_loop(0, (_NCH + 3) // 4, iter4, 0)
    wait_scatter((_NCH - 1) % 4 % nbuf, (_NCH - 1) % 4)
    plsc.subcore_barrier()
    pltpu.sync_copy(acc.at[pl.ds(r0, _RPT)], agg_o.at[c, pl.ds(r0, _RPT)])
    if with_cnt:
      pltpu.sync_copy(cacc.at[pl.ds(r0, _RPT)], cnt_o.at[c, pl.ds(r0, _RPT)])

  outs = [jax.ShapeDtypeStruct((_NC, _NPAD, d), jnp.float32)]
  inputs = [feat, idx3, jnp.zeros((_NPAD, d), jnp.float32)]
  scratch = (
      [pltpu.VMEM((2, _C), jnp.int32) for _ in range(4)]
      + [pltpu.VMEM((_C, d), jnp.float32) for _ in range(nbuf)]
  )
  if stage:
    scratch += [pltpu.VMEM_SHARED((_N, d), jnp.float32)]
  if with_cnt:
    outs.append(jax.ShapeDtypeStruct((_NC, _NPAD, 8), jnp.float32))
    inputs += [jnp.ones((_C, 8), jnp.float32),
               jnp.zeros((_NPAD, 8), jnp.float32)]
    scratch += [pltpu.VMEM_SHARED((_NPAD, d), jnp.float32),
                pltpu.VMEM((_C, 8), jnp.float32),
                pltpu.VMEM_SHARED((_NPAD, 8), jnp.float32)]
  else:
    scratch += [pltpu.VMEM_SHARED((_NPAD, d), jnp.float32)]
  nsem = (4 + 3 * nbuf) if with_cnt else (4 + 2 * nbuf)
  scratch += [pltpu.SemaphoreType.DMA for _ in range(nsem)]

  f = pl.kernel(
      body,
      out_type=tuple(outs),
      mesh=_mesh,
      compiler_params=pltpu.CompilerParams(use_tc_tiling_on_sc=False),
      scratch_types=tuple(scratch),
  )
  return f(*inputs)


def _dot_t(a, w):
  return lax.dot_general(a, w, (((1,), (1,)), ((), ())),
                         precision=lax.Precision.HIGHEST,
                         preferred_element_type=jnp.float32)


def _mean(agg_r, cnt_r):
  cnt = cnt_r[0, :, 0:1] + cnt_r[1, :, 0:1]
  inv = 1.0 / jnp.maximum(cnt, 1.0)
  return (agg_r[0] + agg_r[1]) * inv


def _elu(h):
  return jnp.where(h > 0, h, jnp.exp(jnp.minimum(h, 0.0)) - 1.0)


_BM = 1000


def _t1_body(x_r, a_r, c_r, w1l_r, b1_r, w1r_r, w2l_r, b2_r, w2r_r,
             p2_o, r2_o):
  mean = _mean(a_r, c_r)
  h1 = _dot_t(mean, w1l_r[...]) + b1_r[...] + _dot_t(x_r[...], w1r_r[...])
  h1 = _elu(h1)
  p2_o[...] = _dot_t(h1, w2l_r[...])
  r2_o[...] = _dot_t(h1, w2r_r[...]) + b2_r[...]


def _t1(x, agg1, cnt, w1l, b1, w1r, w2l, b2, w2r):
  grid = (_N // _BM,)
  full = lambda shape: pl.BlockSpec(shape, lambda i: (0,) * len(shape))
  return pl.pallas_call(
      _t1_body,
      grid=grid,
      in_specs=[
          pl.BlockSpec((_BM, 128), lambda i: (i, 0)),
          pl.BlockSpec((_NC, _BM, 128), lambda i: (0, i, 0)),
          pl.BlockSpec((_NC, _BM, 8), lambda i: (0, i, 0)),
          full((256, 128)), full((1, 256)), full((256, 128)),
          full((32, 256)), full((1, 32)), full((32, 256)),
      ],
      out_specs=[
          pl.BlockSpec((_BM, 32), lambda i: (i, 0)),
          pl.BlockSpec((_BM, 32), lambda i: (i, 0)),
      ],
      out_shape=[
          jax.ShapeDtypeStruct((_N, 32), jnp.float32),
          jax.ShapeDtypeStruct((_N, 32), jnp.float32),
      ],
  )(x, agg1, cnt, w1l, b1, w1r, w2l, b2, w2r)


def _t2_body(a_r, c_r, r2_r, h2_o):
  h2_o[...] = _elu(_mean(a_r, c_r) + r2_r[...])


def _t2(agg2, cnt, r2):
  grid = (_N // _BM,)
  return pl.pallas_call(
      _t2_body,
      grid=grid,
      in_specs=[
          pl.BlockSpec((_NC, _BM, 32), lambda i: (0, i, 0)),
          pl.BlockSpec((_NC, _BM, 8), lambda i: (0, i, 0)),
          pl.BlockSpec((_BM, 32), lambda i: (i, 0)),
      ],
      out_specs=pl.BlockSpec((_BM, 32), lambda i: (i, 0)),
      out_shape=jax.ShapeDtypeStruct((_N, 32), jnp.float32),
  )(agg2, cnt, r2)


def _t3_body(a_r, c_r, h2_r, w3l_r, b3_r, w3r_r, out_o):
  mean = _mean(a_r, c_r)
  out_o[...] = (_dot_t(mean, w3l_r[...]) + b3_r[...]
                + _dot_t(h2_r[...], w3r_r[...]))


def _t3(agg3, cnt, h2, w3l, b3, w3r):
  grid = (_N // _BM,)
  full = lambda shape: pl.BlockSpec(shape, lambda i: (0,) * len(shape))
  return pl.pallas_call(
      _t3_body,
      grid=grid,
      in_specs=[
          pl.BlockSpec((_NC, _BM, 32), lambda i: (0, i, 0)),
          pl.BlockSpec((_NC, _BM, 8), lambda i: (0, i, 0)),
          pl.BlockSpec((_BM, 32), lambda i: (i, 0)),
          full((64, 32)), full((1, 64)), full((64, 32)),
      ],
      out_specs=pl.BlockSpec((_BM, 64), lambda i: (i, 0)),
      out_shape=jax.ShapeDtypeStruct((_N, 64), jnp.float32),
  )(agg3, cnt, h2, w3l, b3, w3r)


def kernel(x, edge_index, W1l, b1, W1r, W2l, b2, W2r, W3l, b3, W3r):
  ei = edge_index.astype(jnp.int32)
  src = jnp.pad(ei[0], (0, _EPAD - _E))
  dst = jnp.pad(ei[1], (0, _EPAD - _E), constant_values=_PAD_NODE)
  idx3 = jnp.stack([src.reshape(_NW, _NCH, _C),
                    dst.reshape(_NW, _NCH, _C)], axis=2)

  xp = jnp.pad(x, ((0, _NPAD - _N), (0, 0)))
  agg1, cnt = _seg_sum(xp, idx3, 128, True)
  p2, r2 = _t1(x, agg1, cnt, W1l, b1.reshape(1, 256), W1r,
               W2l, b2.reshape(1, 32), W2r)
  agg2 = _seg_sum(p2, idx3, 32, False)[0]
  h2 = _t2(agg2, cnt, r2)
  agg3 = _seg_sum(h2, idx3, 32, False)[0]
  return _t3(agg3, cnt, h2, W3l, b3.reshape(1, 64), W3r)


# speedup vs baseline: 1.3546x; 1.1999x over previous
import functools

import jax
import jax.numpy as jnp
from jax import lax
from jax.experimental import pallas as pl
from jax.experimental.pallas import tpu as pltpu
from jax.experimental.pallas import tpu_sc as plsc

_N = 10000
_E = 320000
_NC = 2
_NS = 16
_NW = _NC * _NS
_C = 128
_NCH = -(-_E // (_NW * _C))
_EPAD = _NW * _NCH * _C
_RPT = 640
_NPAD = _RPT * _NS
_PAD_NODE = _N + 16

_mesh = plsc.VectorSubcoreMesh(core_axis_name="c", subcore_axis_name="s")


def _seg_sum(feat, idx3, col0, d, with_cnt):
  dful = feat.shape[1]
  nbuf = 2 if d > 32 else 4
  depth = 1 if nbuf == 2 else 2
  stage = True
  srows = _N // _NS

  def body(*refs):
    if with_cnt:
      (feat_h, idx_h, zf_h, ones_h, zc_h, agg_o, cnt_o) = refs[:7]
      refs = refs[7:]
    else:
      (feat_h, idx_h, zf_h, agg_o) = refs[:4]
      refs = refs[4:]
    idx2 = list(refs[:4])
    rows = list(refs[4:4 + nbuf])
    refs = refs[4 + nbuf:]
    if stage:
      feat_s = refs[0]
      refs = refs[1:]
    else:
      feat_s = None
    if with_cnt:
      acc, ones_v, cacc = refs[:3]
      refs = refs[3:]
    else:
      acc = refs[0]
      refs = refs[1:]
    isem = list(refs[:4])
    gsem = list(refs[4:4 + nbuf])
    ssem = list(refs[4 + nbuf:4 + 2 * nbuf])
    if with_cnt:
      csem = list(refs[4 + 2 * nbuf:4 + 3 * nbuf])

    c = lax.axis_index("c")
    s = lax.axis_index("s")
    wid = c * _NS + s
    r0 = s * _RPT

    def idx_load(j, q):
      return pltpu.async_copy(idx_h.at[wid, j], idx2[q], isem[q])

    feat_src = feat_s if stage else feat_h

    def gather(j, q, b):
      return pltpu.async_copy(feat_src.at[idx2[q].at[0]], rows[b], gsem[b])

    def scatter(b, q):
      pltpu.async_copy(rows[b], acc.at[idx2[q].at[1]], ssem[b], add=True)
      if with_cnt:
        pltpu.async_copy(ones_v, cacc.at[idx2[q].at[1]], csem[b], add=True)

    def wait_scatter(b, q):
      pltpu.make_async_copy(rows[b], acc.at[idx2[q].at[1]], ssem[b]).wait()
      if with_cnt:
        pltpu.make_async_copy(ones_v, cacc.at[idx2[q].at[1]], csem[b]).wait()

    pltpu.sync_copy(zf_h.at[pl.ds(r0, _RPT)], acc.at[pl.ds(r0, _RPT)])
    if stage:
      if d == dful:
        pltpu.sync_copy(feat_h.at[pl.ds(s * srows, srows)],
                        feat_s.at[pl.ds(s * srows, srows)])
      else:
        pltpu.sync_copy(feat_h.at[pl.ds(s * srows, srows), pl.ds(col0, d)],
                        feat_s.at[pl.ds(s * srows, srows)])
    if with_cnt:
      pltpu.sync_copy(zc_h.at[pl.ds(r0, _RPT)], cacc.at[pl.ds(r0, _RPT)])
      pltpu.sync_copy(ones_h, ones_v)
    descs = [idx_load(t, t) for t in range(3)]
    plsc.subcore_barrier()
    for t in range(depth):
      descs[t].wait()
      gather(t, t, t % nbuf)

    def iter4(jj, carry):
      for q in range(4):
        j = jj * 4 + q
        b = q % nbuf

        @pl.when(j < _NCH)
        def _():
          pltpu.make_async_copy(feat_src.at[idx2[q].at[0]], rows[b],
                                gsem[b]).wait()
          scatter(b, q)

          @pl.when(j >= 1)
          def _():
            wait_scatter((q + 3) % 4 % nbuf, (q + 3) % 4)

          @pl.when(j + depth < _NCH)
          def _():
            pltpu.make_async_copy(idx_h.at[wid, 0], idx2[(q + depth) % 4],
                                  isem[(q + depth) % 4]).wait()
            gather(j + depth, (q + depth) % 4, (q + depth) % 4 % nbuf)

          @pl.when(j + 3 < _NCH)
          def _():
            idx_load(j + 3, (q + 3) % 4)
      return carry

    lax.fori_loop(0, (_NCH + 3) // 4, iter4, 0)
    wait_scatter((_NCH - 1) % 4 % nbuf, (_NCH - 1) % 4)
    plsc.subcore_barrier()
    pltpu.sync_copy(acc.at[pl.ds(r0, _RPT)], agg_o.at[c, pl.ds(r0, _RPT)])
    if with_cnt:
      pltpu.sync_copy(cacc.at[pl.ds(r0, _RPT)], cnt_o.at[c, pl.ds(r0, _RPT)])

  outs = [jax.ShapeDtypeStruct((_NC, _NPAD, d), jnp.float32)]
  inputs = [feat, idx3, jnp.zeros((_NPAD, d), jnp.float32)]
  scratch = (
      [pltpu.VMEM((2, _C), jnp.int32) for _ in range(4)]
      + [pltpu.VMEM((_C, d), jnp.float32) for _ in range(nbuf)]
  )
  if stage:
    scratch += [pltpu.VMEM_SHARED((_N, d), jnp.float32)]
  if with_cnt:
    outs.append(jax.ShapeDtypeStruct((_NC, _NPAD, 8), jnp.float32))
    inputs += [jnp.ones((_C, 8), jnp.float32),
               jnp.zeros((_NPAD, 8), jnp.float32)]
    scratch += [pltpu.VMEM_SHARED((_NPAD, d), jnp.float32),
                pltpu.VMEM((_C, 8), jnp.float32),
                pltpu.VMEM_SHARED((_NPAD, 8), jnp.float32)]
  else:
    scratch += [pltpu.VMEM_SHARED((_NPAD, d), jnp.float32)]
  nsem = (4 + 3 * nbuf) if with_cnt else (4 + 2 * nbuf)
  scratch += [pltpu.SemaphoreType.DMA for _ in range(nsem)]

  f = pl.kernel(
      body,
      out_type=tuple(outs),
      mesh=_mesh,
      compiler_params=pltpu.CompilerParams(use_tc_tiling_on_sc=False),
      scratch_types=tuple(scratch),
  )
  return f(*inputs)


def _dot_t(a, w):
  return lax.dot_general(a, w, (((1,), (1,)), ((), ())),
                         precision=lax.Precision.HIGHEST,
                         preferred_element_type=jnp.float32)


def _mean(agg_r, cnt_r):
  cnt = cnt_r[0, :, 0:1] + cnt_r[1, :, 0:1]
  inv = 1.0 / jnp.maximum(cnt, 1.0)
  return (agg_r[0] + agg_r[1]) * inv


def _elu(h):
  return jnp.where(h > 0, h, jnp.exp(jnp.minimum(h, 0.0)) - 1.0)


_BM = 1000


def _t1_body(x_r, aa_r, ab_r, c_r, w1la_r, w1lb_r, b1_r, w1r_r,
             w2l_r, b2_r, w2r_r, p2_o, r2_o):
  cnt = c_r[0, :, 0:1] + c_r[1, :, 0:1]
  inv = 1.0 / jnp.maximum(cnt, 1.0)
  mean_a = (aa_r[0] + aa_r[1]) * inv
  mean_b = (ab_r[0] + ab_r[1]) * inv
  h1 = (_dot_t(mean_a, w1la_r[...]) + _dot_t(mean_b, w1lb_r[...])
        + b1_r[...] + _dot_t(x_r[...], w1r_r[...]))
  h1 = _elu(h1)
  p2_o[...] = _dot_t(h1, w2l_r[...])
  r2_o[...] = _dot_t(h1, w2r_r[...]) + b2_r[...]


def _t1(x, agg1a, agg1b, cnt, w1l, b1, w1r, w2l, b2, w2r):
  grid = (_N // _BM,)
  full = lambda shape: pl.BlockSpec(shape, lambda i: (0,) * len(shape))
  return pl.pallas_call(
      _t1_body,
      grid=grid,
      in_specs=[
          pl.BlockSpec((_BM, 128), lambda i: (i, 0)),
          pl.BlockSpec((_NC, _BM, 64), lambda i: (0, i, 0)),
          pl.BlockSpec((_NC, _BM, 64), lambda i: (0, i, 0)),
          pl.BlockSpec((_NC, _BM, 8), lambda i: (0, i, 0)),
          full((256, 64)), full((256, 64)), full((1, 256)), full((256, 128)),
          full((32, 256)), full((1, 32)), full((32, 256)),
      ],
      out_specs=[
          pl.BlockSpec((_BM, 32), lambda i: (i, 0)),
          pl.BlockSpec((_BM, 32), lambda i: (i, 0)),
      ],
      out_shape=[
          jax.ShapeDtypeStruct((_N, 32), jnp.float32),
          jax.ShapeDtypeStruct((_N, 32), jnp.float32),
      ],
  )(x, agg1a, agg1b, cnt, w1l[:, :64], w1l[:, 64:], b1, w1r,
    w2l, b2, w2r)


def _t2_body(a_r, c_r, r2_r, h2_o):
  h2_o[...] = _elu(_mean(a_r, c_r) + r2_r[...])


def _t2(agg2, cnt, r2):
  grid = (_N // _BM,)
  return pl.pallas_call(
      _t2_body,
      grid=grid,
      in_specs=[
          pl.BlockSpec((_NC, _BM, 32), lambda i: (0, i, 0)),
          pl.BlockSpec((_NC, _BM, 8), lambda i: (0, i, 0)),
          pl.BlockSpec((_BM, 32), lambda i: (i, 0)),
      ],
      out_specs=pl.BlockSpec((_BM, 32), lambda i: (i, 0)),
      out_shape=jax.ShapeDtypeStruct((_N, 32), jnp.float32),
  )(agg2, cnt, r2)


def _t3_body(a_r, c_r, h2_r, w3l_r, b3_r, w3r_r, out_o):
  mean = _mean(a_r, c_r)
  out_o[...] = (_dot_t(mean, w3l_r[...]) + b3_r[...]
                + _dot_t(h2_r[...], w3r_r[...]))


def _t3(agg3, cnt, h2, w3l, b3, w3r):
  grid = (_N // _BM,)
  full = lambda shape: pl.BlockSpec(shape, lambda i: (0,) * len(shape))
  return pl.pallas_call(
      _t3_body,
      grid=grid,
      in_specs=[
          pl.BlockSpec((_NC, _BM, 32), lambda i: (0, i, 0)),
          pl.BlockSpec((_NC, _BM, 8), lambda i: (0, i, 0)),
          pl.BlockSpec((_BM, 32), lambda i: (i, 0)),
          full((64, 32)), full((1, 64)), full((64, 32)),
      ],
      out_specs=pl.BlockSpec((_BM, 64), lambda i: (i, 0)),
      out_shape=jax.ShapeDtypeStruct((_N, 64), jnp.float32),
  )(agg3, cnt, h2, w3l, b3, w3r)


def kernel(x, edge_index, W1l, b1, W1r, W2l, b2, W2r, W3l, b3, W3r):
  ei = edge_index.astype(jnp.int32)
  src = jnp.pad(ei[0], (0, _EPAD - _E))
  dst = jnp.pad(ei[1], (0, _EPAD - _E), constant_values=_PAD_NODE)
  idx3 = jnp.stack([src.reshape(_NW, _NCH, _C),
                    dst.reshape(_NW, _NCH, _C)], axis=2)

  agg1a, cnt = _seg_sum(x, idx3, 0, 64, True)
  agg1b = _seg_sum(x, idx3, 64, 64, False)[0]
  p2, r2 = _t1(x, agg1a, agg1b, cnt, W1l, b1.reshape(1, 256), W1r,
               W2l, b2.reshape(1, 32), W2r)
  agg2 = _seg_sum(p2, idx3, 0, 32, False)[0]
  h2 = _t2(agg2, cnt, r2)
  agg3 = _seg_sum(h2, idx3, 0, 32, False)[0]
  return _t3(agg3, cnt, h2, W3l, b3.reshape(1, 64), W3r)


# speedup vs baseline: 1.6304x; 1.2036x over previous
import functools

import jax
import jax.numpy as jnp
from jax import lax
from jax.experimental import pallas as pl
from jax.experimental.pallas import tpu as pltpu
from jax.experimental.pallas import tpu_sc as plsc

_N = 10000
_E = 320000
_NC = 2
_NS = 16
_NW = _NC * _NS
_C = 128
_NCH = -(-_E // (_NW * _C))
_EPAD = _NW * _NCH * _C
_RPT = 640
_NPAD = _RPT * _NS
_PAD_NODE = _N + 16

_mesh = plsc.VectorSubcoreMesh(core_axis_name="c", subcore_axis_name="s")


def _seg_sum(feat, idx3, col0, d, with_cnt):
  dful = feat.shape[1]
  nbuf = 2 if d > 32 else 4
  depth = 1 if nbuf == 2 else 2
  stage = True
  srows = _N // _NS

  def body(*refs):
    if with_cnt:
      (feat_h, idx_h, zf_h, ones_h, zc_h, agg_o, cnt_o) = refs[:7]
      refs = refs[7:]
    else:
      (feat_h, idx_h, zf_h, agg_o) = refs[:4]
      refs = refs[4:]
    idx2 = list(refs[:4])
    rows = list(refs[4:4 + nbuf])
    refs = refs[4 + nbuf:]
    if stage:
      feat_s = refs[0]
      refs = refs[1:]
    else:
      feat_s = None
    if with_cnt:
      acc, ones_v, cacc = refs[:3]
      refs = refs[3:]
    else:
      acc = refs[0]
      refs = refs[1:]
    isem = list(refs[:4])
    gsem = list(refs[4:4 + nbuf])
    ssem = list(refs[4 + nbuf:4 + 2 * nbuf])
    if with_cnt:
      csem = list(refs[4 + 2 * nbuf:4 + 3 * nbuf])

    c = lax.axis_index("c")
    s = lax.axis_index("s")
    wid = c * _NS + s
    r0 = s * _RPT

    def idx_load(j, q):
      return pltpu.async_copy(idx_h.at[wid, j], idx2[q], isem[q])

    feat_src = feat_s if stage else feat_h

    def gather(j, q, b):
      return pltpu.async_copy(feat_src.at[idx2[q].at[0]], rows[b], gsem[b])

    def scatter(b, q):
      pltpu.async_copy(rows[b], acc.at[idx2[q].at[1]], ssem[b], add=True)
      if with_cnt:
        pltpu.async_copy(ones_v, cacc.at[idx2[q].at[1]], csem[b], add=True)

    def wait_scatter(b, q):
      pltpu.make_async_copy(rows[b], acc.at[idx2[q].at[1]], ssem[b]).wait()
      if with_cnt:
        pltpu.make_async_copy(ones_v, cacc.at[idx2[q].at[1]], csem[b]).wait()

    pltpu.sync_copy(zf_h.at[pl.ds(r0, _RPT)], acc.at[pl.ds(r0, _RPT)])
    if stage:
      if d == dful:
        pltpu.sync_copy(feat_h.at[pl.ds(s * srows, srows)],
                        feat_s.at[pl.ds(s * srows, srows)])
      else:
        pltpu.sync_copy(feat_h.at[pl.ds(s * srows, srows), pl.ds(col0, d)],
                        feat_s.at[pl.ds(s * srows, srows)])
    if with_cnt:
      pltpu.sync_copy(zc_h.at[pl.ds(r0, _RPT)], cacc.at[pl.ds(r0, _RPT)])
      pltpu.sync_copy(ones_h, ones_v)
    descs = [idx_load(t, t) for t in range(3)]
    plsc.subcore_barrier()
    for t in range(depth):
      descs[t].wait()
      gather(t, t, t % nbuf)

    def iter4(jj, carry):
      for q in range(4):
        j = jj * 4 + q
        b = q % nbuf

        @pl.when(j < _NCH)
        def _():
          pltpu.make_async_copy(feat_src.at[idx2[q].at[0]], rows[b],
                                gsem[b]).wait()
          scatter(b, q)

          @pl.when(j >= 1)
          def _():
            wait_scatter((q + 3) % 4 % nbuf, (q + 3) % 4)

          @pl.when(j + depth < _NCH)
          def _():
            pltpu.make_async_copy(idx_h.at[wid, 0], idx2[(q + depth) % 4],
                                  isem[(q + depth) % 4]).wait()
            gather(j + depth, (q + depth) % 4, (q + depth) % 4 % nbuf)

          @pl.when(j + 3 < _NCH)
          def _():
            idx_load(j + 3, (q + 3) % 4)
      return carry

    lax.fori_loop(0, (_NCH + 3) // 4, iter4, 0)
    wait_scatter((_NCH - 1) % 4 % nbuf, (_NCH - 1) % 4)
    plsc.subcore_barrier()
    pltpu.sync_copy(acc.at[pl.ds(r0, _RPT)], agg_o.at[c, pl.ds(r0, _RPT)])
    if with_cnt:
      pltpu.sync_copy(cacc.at[pl.ds(r0, _RPT)], cnt_o.at[c, pl.ds(r0, _RPT)])

  outs = [jax.ShapeDtypeStruct((_NC, _NPAD, d), jnp.float32)]
  inputs = [feat, idx3, jnp.zeros((_NPAD, d), jnp.float32)]
  scratch = (
      [pltpu.VMEM((2, _C), jnp.int32) for _ in range(4)]
      + [pltpu.VMEM((_C, d), jnp.float32) for _ in range(nbuf)]
  )
  if stage:
    scratch += [pltpu.VMEM_SHARED((_N, d), jnp.float32)]
  if with_cnt:
    outs.append(jax.ShapeDtypeStruct((_NC, _NPAD, 8), jnp.float32))
    inputs += [jnp.ones((_C, 8), jnp.float32),
               jnp.zeros((_NPAD, 8), jnp.float32)]
    scratch += [pltpu.VMEM_SHARED((_NPAD, d), jnp.float32),
                pltpu.VMEM((_C, 8), jnp.float32),
                pltpu.VMEM_SHARED((_NPAD, 8), jnp.float32)]
  else:
    scratch += [pltpu.VMEM_SHARED((_NPAD, d), jnp.float32)]
  nsem = (4 + 3 * nbuf) if with_cnt else (4 + 2 * nbuf)
  scratch += [pltpu.SemaphoreType.DMA for _ in range(nsem)]

  f = pl.kernel(
      body,
      out_type=tuple(outs),
      mesh=_mesh,
      compiler_params=pltpu.CompilerParams(use_tc_tiling_on_sc=False),
      scratch_types=tuple(scratch),
  )
  return f(*inputs)


def _dot_t(a, w):
  return lax.dot_general(a, w, (((1,), (1,)), ((), ())),
                         precision=lax.Precision.DEFAULT,
                         preferred_element_type=jnp.float32)


def _mean(agg_r, cnt_r):
  cnt = cnt_r[0, :, 0:1] + cnt_r[1, :, 0:1]
  inv = 1.0 / jnp.maximum(cnt, 1.0)
  return (agg_r[0] + agg_r[1]) * inv


def _elu(h):
  return jnp.where(h > 0, h, jnp.exp(jnp.minimum(h, 0.0)) - 1.0)


_BM = 1000


def _t1_body(x_r, aa_r, ab_r, c_r, w1la_r, w1lb_r, b1_r, w1r_r,
             w2l_r, b2_r, w2r_r, p2_o, r2_o):
  cnt = c_r[0, :, 0:1] + c_r[1, :, 0:1]
  inv = 1.0 / jnp.maximum(cnt, 1.0)
  mean_a = (aa_r[0] + aa_r[1]) * inv
  mean_b = (ab_r[0] + ab_r[1]) * inv
  h1 = (_dot_t(mean_a, w1la_r[...]) + _dot_t(mean_b, w1lb_r[...])
        + b1_r[...] + _dot_t(x_r[...], w1r_r[...]))
  h1 = _elu(h1)
  p2_o[...] = _dot_t(h1, w2l_r[...])
  r2_o[...] = _dot_t(h1, w2r_r[...]) + b2_r[...]


def _t1(x, agg1a, agg1b, cnt, w1l, b1, w1r, w2l, b2, w2r):
  grid = (_N // _BM,)
  full = lambda shape: pl.BlockSpec(shape, lambda i: (0,) * len(shape))
  return pl.pallas_call(
      _t1_body,
      grid=grid,
      in_specs=[
          pl.BlockSpec((_BM, 128), lambda i: (i, 0)),
          pl.BlockSpec((_NC, _BM, 64), lambda i: (0, i, 0)),
          pl.BlockSpec((_NC, _BM, 64), lambda i: (0, i, 0)),
          pl.BlockSpec((_NC, _BM, 8), lambda i: (0, i, 0)),
          full((256, 64)), full((256, 64)), full((1, 256)), full((256, 128)),
          full((32, 256)), full((1, 32)), full((32, 256)),
      ],
      out_specs=[
          pl.BlockSpec((_BM, 32), lambda i: (i, 0)),
          pl.BlockSpec((_BM, 32), lambda i: (i, 0)),
      ],
      out_shape=[
          jax.ShapeDtypeStruct((_N, 32), jnp.float32),
          jax.ShapeDtypeStruct((_N, 32), jnp.float32),
      ],
  )(x, agg1a, agg1b, cnt, w1l[:, :64], w1l[:, 64:], b1, w1r,
    w2l, b2, w2r)


def _t2_body(a_r, c_r, r2_r, h2_o):
  h2_o[...] = _elu(_mean(a_r, c_r) + r2_r[...])


def _t2(agg2, cnt, r2):
  grid = (_N // _BM,)
  return pl.pallas_call(
      _t2_body,
      grid=grid,
      in_specs=[
          pl.BlockSpec((_NC, _BM, 32), lambda i: (0, i, 0)),
          pl.BlockSpec((_NC, _BM, 8), lambda i: (0, i, 0)),
          pl.BlockSpec((_BM, 32), lambda i: (i, 0)),
      ],
      out_specs=pl.BlockSpec((_BM, 32), lambda i: (i, 0)),
      out_shape=jax.ShapeDtypeStruct((_N, 32), jnp.float32),
  )(agg2, cnt, r2)


def _t3_body(a_r, c_r, h2_r, w3l_r, b3_r, w3r_r, out_o):
  mean = _mean(a_r, c_r)
  out_o[...] = (_dot_t(mean, w3l_r[...]) + b3_r[...]
                + _dot_t(h2_r[...], w3r_r[...]))


def _t3(agg3, cnt, h2, w3l, b3, w3r):
  grid = (_N // _BM,)
  full = lambda shape: pl.BlockSpec(shape, lambda i: (0,) * len(shape))
  return pl.pallas_call(
      _t3_body,
      grid=grid,
      in_specs=[
          pl.BlockSpec((_NC, _BM, 32), lambda i: (0, i, 0)),
          pl.BlockSpec((_NC, _BM, 8), lambda i: (0, i, 0)),
          pl.BlockSpec((_BM, 32), lambda i: (i, 0)),
          full((64, 32)), full((1, 64)), full((64, 32)),
      ],
      out_specs=pl.BlockSpec((_BM, 64), lambda i: (i, 0)),
      out_shape=jax.ShapeDtypeStruct((_N, 64), jnp.float32),
  )(agg3, cnt, h2, w3l, b3, w3r)


def kernel(x, edge_index, W1l, b1, W1r, W2l, b2, W2r, W3l, b3, W3r):
  ei = edge_index.astype(jnp.int32)
  src = jnp.pad(ei[0], (0, _EPAD - _E))
  dst = jnp.pad(ei[1], (0, _EPAD - _E), constant_values=_PAD_NODE)
  idx3 = jnp.stack([src.reshape(_NW, _NCH, _C),
                    dst.reshape(_NW, _NCH, _C)], axis=2)

  agg1a, cnt = _seg_sum(x, idx3, 0, 64, True)
  agg1b = _seg_sum(x, idx3, 64, 64, False)[0]
  p2, r2 = _t1(x, agg1a, agg1b, cnt, W1l, b1.reshape(1, 256), W1r,
               W2l, b2.reshape(1, 32), W2r)
  agg2 = _seg_sum(p2, idx3, 0, 32, False)[0]
  h2 = _t2(agg2, cnt, r2)
  agg3 = _seg_sum(h2, idx3, 0, 32, False)[0]
  return _t3(agg3, cnt, h2, W3l, b3.reshape(1, 64), W3r)


# speedup vs baseline: 1.6643x; 1.0208x over previous
import functools

import jax
import jax.numpy as jnp
from jax import lax
from jax.experimental import pallas as pl
from jax.experimental.pallas import tpu as pltpu
from jax.experimental.pallas import tpu_sc as plsc

_N = 10000
_E = 320000
_NC = 2
_NS = 16
_NW = _NC * _NS
_C = 128
_NCH = -(-_E // (_NW * _C))
_EPAD = _NW * _NCH * _C
_RPT = 640
_NPAD = _RPT * _NS
_PAD_NODE = _N + 16

_mesh = plsc.VectorSubcoreMesh(core_axis_name="c", subcore_axis_name="s")


def _seg_sum(feat, idx3, col0, d, with_cnt):
  dful = feat.shape[1]
  nbuf = 2 if d > 32 else 4
  depth = 1 if nbuf == 2 else 2
  stage = True
  srows = _N // _NS

  def body(*refs):
    if with_cnt:
      (feat_h, idx_h, zf_h, ones_h, zc_h, agg_o, cnt_o) = refs[:7]
      refs = refs[7:]
    else:
      (feat_h, idx_h, zf_h, agg_o) = refs[:4]
      refs = refs[4:]
    idx2 = list(refs[:4])
    rows = list(refs[4:4 + nbuf])
    refs = refs[4 + nbuf:]
    if stage:
      feat_s = refs[0]
      refs = refs[1:]
    else:
      feat_s = None
    if with_cnt:
      acc, ones_v, cacc = refs[:3]
      refs = refs[3:]
    else:
      acc = refs[0]
      refs = refs[1:]
    isem = list(refs[:4])
    gsem = list(refs[4:4 + nbuf])
    ssem = list(refs[4 + nbuf:4 + 2 * nbuf])
    if with_cnt:
      csem = list(refs[4 + 2 * nbuf:4 + 3 * nbuf])

    c = lax.axis_index("c")
    s = lax.axis_index("s")
    wid = c * _NS + s
    r0 = s * _RPT

    def idx_load(j, q):
      return pltpu.async_copy(idx_h.at[wid, j], idx2[q], isem[q])

    feat_src = feat_s if stage else feat_h

    def gather(j, q, b):
      return pltpu.async_copy(feat_src.at[idx2[q].at[0]], rows[b], gsem[b])

    def scatter(b, q):
      pltpu.async_copy(rows[b], acc.at[idx2[q].at[1]], ssem[b], add=True)
      if with_cnt:
        pltpu.async_copy(ones_v, cacc.at[idx2[q].at[1]], csem[b], add=True)

    def wait_scatter(b, q):
      pltpu.make_async_copy(rows[b], acc.at[idx2[q].at[1]], ssem[b]).wait()
      if with_cnt:
        pltpu.make_async_copy(ones_v, cacc.at[idx2[q].at[1]], csem[b]).wait()

    pltpu.sync_copy(zf_h.at[pl.ds(r0, _RPT)], acc.at[pl.ds(r0, _RPT)])
    if stage:
      if d == dful:
        pltpu.sync_copy(feat_h.at[pl.ds(s * srows, srows)],
                        feat_s.at[pl.ds(s * srows, srows)])
      else:
        pltpu.sync_copy(feat_h.at[pl.ds(s * srows, srows), pl.ds(col0, d)],
                        feat_s.at[pl.ds(s * srows, srows)])
    if with_cnt:
      pltpu.sync_copy(zc_h.at[pl.ds(r0, _RPT)], cacc.at[pl.ds(r0, _RPT)])
      pltpu.sync_copy(ones_h, ones_v)
    descs = [idx_load(t, t) for t in range(3)]
    plsc.subcore_barrier()
    for t in range(depth):
      descs[t].wait()
      gather(t, t, t % nbuf)

    def iter4(jj, carry):
      for q in range(4):
        j = jj * 4 + q
        b = q % nbuf

        @pl.when(j < _NCH)
        def _():
          pltpu.make_async_copy(feat_src.at[idx2[q].at[0]], rows[b],
                                gsem[b]).wait()
          scatter(b, q)

          @pl.when(j >= 1)
          def _():
            wait_scatter((q + 3) % 4 % nbuf, (q + 3) % 4)

          @pl.when(j + depth < _NCH)
          def _():
            pltpu.make_async_copy(idx_h.at[wid, 0], idx2[(q + depth) % 4],
                                  isem[(q + depth) % 4]).wait()
            gather(j + depth, (q + depth) % 4, (q + depth) % 4 % nbuf)

          @pl.when(j + 3 < _NCH)
          def _():
            idx_load(j + 3, (q + 3) % 4)
      return carry

    lax.fori_loop(0, (_NCH + 3) // 4, iter4, 0)
    wait_scatter((_NCH - 1) % 4 % nbuf, (_NCH - 1) % 4)
    plsc.subcore_barrier()
    pltpu.sync_copy(acc.at[pl.ds(r0, _RPT)], agg_o.at[c, pl.ds(r0, _RPT)])
    if with_cnt:
      pltpu.sync_copy(cacc.at[pl.ds(r0, _RPT)], cnt_o.at[c, pl.ds(r0, _RPT)])

  outs = [jax.ShapeDtypeStruct((_NC, _NPAD, d), jnp.float32)]
  inputs = [feat, idx3, jnp.zeros((_NPAD, d), jnp.float32)]
  scratch = (
      [pltpu.VMEM((2, _C), jnp.int32) for _ in range(4)]
      + [pltpu.VMEM((_C, d), jnp.float32) for _ in range(nbuf)]
  )
  if stage:
    scratch += [pltpu.VMEM_SHARED((_N, d), jnp.float32)]
  if with_cnt:
    outs.append(jax.ShapeDtypeStruct((_NC, _NPAD, 8), jnp.float32))
    inputs += [jnp.ones((_C, 8), jnp.float32),
               jnp.zeros((_NPAD, 8), jnp.float32)]
    scratch += [pltpu.VMEM_SHARED((_NPAD, d), jnp.float32),
                pltpu.VMEM((_C, 8), jnp.float32),
                pltpu.VMEM_SHARED((_NPAD, 8), jnp.float32)]
  else:
    scratch += [pltpu.VMEM_SHARED((_NPAD, d), jnp.float32)]
  nsem = (4 + 3 * nbuf) if with_cnt else (4 + 2 * nbuf)
  scratch += [pltpu.SemaphoreType.DMA for _ in range(nsem)]

  f = pl.kernel(
      body,
      out_type=tuple(outs),
      mesh=_mesh,
      compiler_params=pltpu.CompilerParams(use_tc_tiling_on_sc=False),
      scratch_types=tuple(scratch),
  )
  return f(*inputs)


def _dot_t(a, w):
  return lax.dot_general(a, w, (((1,), (1,)), ((), ())),
                         precision=lax.Precision.DEFAULT,
                         preferred_element_type=jnp.float32)


def _mean(agg_r, cnt_r):
  cnt = cnt_r[0, :, 0:1] + cnt_r[1, :, 0:1]
  inv = 1.0 / jnp.maximum(cnt, 1.0)
  return (agg_r[0] + agg_r[1]) * inv


def _elu(h):
  return jnp.where(h > 0, h, jnp.exp(jnp.minimum(h, 0.0)) - 1.0)


_BM = 1000


def _t1_body(x_r, aa_r, ab_r, w1la_r, w1lb_r, b1_r, w1r_r,
             w2l_r, b2_r, w2r_r, p2_o, r2_o, inv_o):
  cnt = aa_r[0, :, 64:65] + aa_r[1, :, 64:65]
  inv = 1.0 / jnp.maximum(cnt, 1.0)
  mean_a = (aa_r[0, :, 0:64] + aa_r[1, :, 0:64]) * inv
  mean_b = (ab_r[0] + ab_r[1]) * inv
  h1 = (_dot_t(mean_a, w1la_r[...]) + _dot_t(mean_b, w1lb_r[...])
        + b1_r[...] + _dot_t(x_r[...], w1r_r[...]))
  h1 = _elu(h1)
  p2_o[...] = _dot_t(h1, w2l_r[...])
  r2_o[...] = _dot_t(h1, w2r_r[...]) + b2_r[...]
  inv_o[...] = jnp.broadcast_to(inv, inv_o.shape)


def _t1(x, agg1a, agg1b, w1l, b1, w1r, w2l, b2, w2r):
  grid = (_N // _BM,)
  full = lambda shape: pl.BlockSpec(shape, lambda i: (0,) * len(shape))
  return pl.pallas_call(
      _t1_body,
      grid=grid,
      in_specs=[
          pl.BlockSpec((_BM, 128), lambda i: (i, 0)),
          pl.BlockSpec((_NC, _BM, 72), lambda i: (0, i, 0)),
          pl.BlockSpec((_NC, _BM, 64), lambda i: (0, i, 0)),
          full((256, 64)), full((256, 64)), full((1, 256)), full((256, 128)),
          full((32, 256)), full((1, 32)), full((32, 256)),
      ],
      out_specs=[
          pl.BlockSpec((_BM, 32), lambda i: (i, 0)),
          pl.BlockSpec((_BM, 32), lambda i: (i, 0)),
          pl.BlockSpec((_BM, 8), lambda i: (i, 0)),
      ],
      out_shape=[
          jax.ShapeDtypeStruct((_N, 32), jnp.float32),
          jax.ShapeDtypeStruct((_N, 32), jnp.float32),
          jax.ShapeDtypeStruct((_N, 8), jnp.float32),
      ],
  )(x, agg1a, agg1b, w1l[:, :64], w1l[:, 64:], b1, w1r,
    w2l, b2, w2r)


def _t2_body(a_r, i_r, r2_r, h2_o):
  mean = (a_r[0] + a_r[1]) * i_r[:, 0:1]
  h2_o[...] = _elu(mean + r2_r[...])


def _t2(agg2, inv, r2):
  grid = (_N // _BM,)
  return pl.pallas_call(
      _t2_body,
      grid=grid,
      in_specs=[
          pl.BlockSpec((_NC, _BM, 32), lambda i: (0, i, 0)),
          pl.BlockSpec((_BM, 8), lambda i: (i, 0)),
          pl.BlockSpec((_BM, 32), lambda i: (i, 0)),
      ],
      out_specs=pl.BlockSpec((_BM, 32), lambda i: (i, 0)),
      out_shape=jax.ShapeDtypeStruct((_N, 32), jnp.float32),
  )(agg2, inv, r2)


def _t3_body(a_r, i_r, h2_r, w3l_r, b3_r, w3r_r, out_o):
  mean = (a_r[0] + a_r[1]) * i_r[:, 0:1]
  out_o[...] = (_dot_t(mean, w3l_r[...]) + b3_r[...]
                + _dot_t(h2_r[...], w3r_r[...]))


def _t3(agg3, inv, h2, w3l, b3, w3r):
  grid = (_N // _BM,)
  full = lambda shape: pl.BlockSpec(shape, lambda i: (0,) * len(shape))
  return pl.pallas_call(
      _t3_body,
      grid=grid,
      in_specs=[
          pl.BlockSpec((_NC, _BM, 32), lambda i: (0, i, 0)),
          pl.BlockSpec((_BM, 8), lambda i: (i, 0)),
          pl.BlockSpec((_BM, 32), lambda i: (i, 0)),
          full((64, 32)), full((1, 64)), full((64, 32)),
      ],
      out_specs=pl.BlockSpec((_BM, 64), lambda i: (i, 0)),
      out_shape=jax.ShapeDtypeStruct((_N, 64), jnp.float32),
  )(agg3, inv, h2, w3l, b3, w3r)


def kernel(x, edge_index, W1l, b1, W1r, W2l, b2, W2r, W3l, b3, W3r):
  ei = edge_index.astype(jnp.int32)
  src = jnp.pad(ei[0], (0, _EPAD - _E))
  dst = jnp.pad(ei[1], (0, _EPAD - _E), constant_values=_PAD_NODE)
  idx3 = jnp.stack([src.reshape(_NW, _NCH, _C),
                    dst.reshape(_NW, _NCH, _C)], axis=2)

  xa = jnp.concatenate([x[:, :64], jnp.ones((_N, 8), jnp.float32)], axis=1)

  agg1a = _seg_sum(xa, idx3, 0, 72, False)[0]
  agg1b = _seg_sum(x, idx3, 64, 64, False)[0]
  p2, r2, inv = _t1(x, agg1a, agg1b, W1l, b1.reshape(1, 256), W1r,
                    W2l, b2.reshape(1, 32), W2r)
  agg2 = _seg_sum(p2, idx3, 0, 32, False)[0]
  h2 = _t2(agg2, inv, r2)
  agg3 = _seg_sum(h2, idx3, 0, 32, False)[0]
  return _t3(agg3, inv, h2, W3l, b3.reshape(1, 64), W3r)


# speedup vs baseline: 1.6680x; 1.0022x over previous
import jax
import jax.numpy as jnp
from jax import lax
from jax.experimental import pallas as pl
from jax.experimental.pallas import tpu as pltpu
from jax.experimental.pallas import tpu_sc as plsc

_N = 10000
_E = 320000
_NC = 2
_NS = 16
_NW = _NC * _NS
_C = 128
_NCH = -(-_E // (_NW * _C))
_EPAD = _NW * _NCH * _C
_RPT = 640
_NPAD = _RPT * _NS
_PAD_NODE = _N + 16

_mesh = plsc.VectorSubcoreMesh(core_axis_name="c", subcore_axis_name="s")


def _seg_sum(feat, idx3, col0, d, with_cnt):
  dful = feat.shape[1]
  nbuf = 2 if d > 32 else 4
  depth = 1 if nbuf == 2 else 2
  stage = True
  srows = _N // _NS

  def body(*refs):
    if with_cnt:
      (feat_h, idx_h, zf_h, ones_h, zc_h, agg_o, cnt_o) = refs[:7]
      refs = refs[7:]
    else:
      (feat_h, idx_h, zf_h, agg_o) = refs[:4]
      refs = refs[4:]
    idx2 = list(refs[:4])
    rows = list(refs[4:4 + nbuf])
    refs = refs[4 + nbuf:]
    if stage:
      feat_s = refs[0]
      refs = refs[1:]
    else:
      feat_s = None
    if with_cnt:
      acc, ones_v, cacc = refs[:3]
      refs = refs[3:]
    else:
      acc = refs[0]
      refs = refs[1:]
    isem = list(refs[:4])
    gsem = list(refs[4:4 + nbuf])
    ssem = list(refs[4 + nbuf:4 + 2 * nbuf])
    if with_cnt:
      csem = list(refs[4 + 2 * nbuf:4 + 3 * nbuf])

    c = lax.axis_index("c")
    s = lax.axis_index("s")
    wid = c * _NS + s
    r0 = s * _RPT

    def idx_load(j, q):
      return pltpu.async_copy(idx_h.at[wid, j], idx2[q], isem[q])

    feat_src = feat_s if stage else feat_h

    def gather(j, q, b):
      return pltpu.async_copy(feat_src.at[idx2[q].at[0]], rows[b], gsem[b])

    def scatter(b, q):
      pltpu.async_copy(rows[b], acc.at[idx2[q].at[1]], ssem[b], add=True)
      if with_cnt:
        pltpu.async_copy(ones_v, cacc.at[idx2[q].at[1]], csem[b], add=True)

    def wait_scatter(b, q):
      pltpu.make_async_copy(rows[b], acc.at[idx2[q].at[1]], ssem[b]).wait()
      if with_cnt:
        pltpu.make_async_copy(ones_v, cacc.at[idx2[q].at[1]], csem[b]).wait()

    pltpu.sync_copy(zf_h.at[pl.ds(r0, _RPT)], acc.at[pl.ds(r0, _RPT)])
    if stage:
      if d == dful:
        pltpu.sync_copy(feat_h.at[pl.ds(s * srows, srows)],
                        feat_s.at[pl.ds(s * srows, srows)])
      else:
        pltpu.sync_copy(feat_h.at[pl.ds(s * srows, srows), pl.ds(col0, d)],
                        feat_s.at[pl.ds(s * srows, srows)])
    if with_cnt:
      pltpu.sync_copy(zc_h.at[pl.ds(r0, _RPT)], cacc.at[pl.ds(r0, _RPT)])
      pltpu.sync_copy(ones_h, ones_v)
    descs = [idx_load(t, t) for t in range(3)]
    plsc.subcore_barrier()
    for t in range(depth):
      descs[t].wait()
      gather(t, t, t % nbuf)

    def iter4(jj, carry):
      for q in range(4):
        j = jj * 4 + q
        b = q % nbuf

        @pl.when(j < _NCH)
        def _():
          pltpu.make_async_copy(feat_src.at[idx2[q].at[0]], rows[b],
                                gsem[b]).wait()
          scatter(b, q)

          @pl.when(j >= 1)
          def _():
            wait_scatter((q + 3) % 4 % nbuf, (q + 3) % 4)

          @pl.when(j + depth < _NCH)
          def _():
            pltpu.make_async_copy(idx_h.at[wid, 0], idx2[(q + depth) % 4],
                                  isem[(q + depth) % 4]).wait()
            gather(j + depth, (q + depth) % 4, (q + depth) % 4 % nbuf)

          @pl.when(j + 3 < _NCH)
          def _():
            idx_load(j + 3, (q + 3) % 4)
      return carry

    lax.fori_loop(0, (_NCH + 3) // 4, iter4, 0)
    wait_scatter((_NCH - 1) % 4 % nbuf, (_NCH - 1) % 4)
    plsc.subcore_barrier()
    pltpu.sync_copy(acc.at[pl.ds(r0, _RPT)], agg_o.at[c, pl.ds(r0, _RPT)])
    if with_cnt:
      pltpu.sync_copy(cacc.at[pl.ds(r0, _RPT)], cnt_o.at[c, pl.ds(r0, _RPT)])

  outs = [jax.ShapeDtypeStruct((_NC, _NPAD, d), jnp.float32)]
  inputs = [feat, idx3, jnp.zeros((_NPAD, d), jnp.float32)]
  scratch = (
      [pltpu.VMEM((2, _C), jnp.int32) for _ in range(4)]
      + [pltpu.VMEM((_C, d), jnp.float32) for _ in range(nbuf)]
  )
  if stage:
    scratch += [pltpu.VMEM_SHARED((_N, d), jnp.float32)]
  if with_cnt:
    outs.append(jax.ShapeDtypeStruct((_NC, _NPAD, 8), jnp.float32))
    inputs += [jnp.ones((_C, 8), jnp.float32),
               jnp.zeros((_NPAD, 8), jnp.float32)]
    scratch += [pltpu.VMEM_SHARED((_NPAD, d), jnp.float32),
                pltpu.VMEM((_C, 8), jnp.float32),
                pltpu.VMEM_SHARED((_NPAD, 8), jnp.float32)]
  else:
    scratch += [pltpu.VMEM_SHARED((_NPAD, d), jnp.float32)]
  nsem = (4 + 3 * nbuf) if with_cnt else (4 + 2 * nbuf)
  scratch += [pltpu.SemaphoreType.DMA for _ in range(nsem)]

  f = pl.kernel(
      body,
      out_type=tuple(outs),
      mesh=_mesh,
      compiler_params=pltpu.CompilerParams(use_tc_tiling_on_sc=False),
      scratch_types=tuple(scratch),
  )
  return f(*inputs)


def _dot_t(a, w):
  return lax.dot_general(a, w, (((1,), (1,)), ((), ())),
                         precision=lax.Precision.DEFAULT,
                         preferred_element_type=jnp.float32)


def _elu(h):
  return jnp.where(h > 0, h, jnp.exp(jnp.minimum(h, 0.0)) - 1.0)


_BM = 1000


def _t1_body(x_r, aa_r, ab_r, w1la_r, w1lb_r, b1_r, w1r_r,
             w2l_r, b2_r, w2r_r, p2_o, r2_o, inv_o):
  cnt = aa_r[0, :, 64:65] + aa_r[1, :, 64:65]
  inv = 1.0 / jnp.maximum(cnt, 1.0)
  mean_a = (aa_r[0, :, 0:64] + aa_r[1, :, 0:64]) * inv
  mean_b = (ab_r[0] + ab_r[1]) * inv
  h1 = (_dot_t(mean_a, w1la_r[...]) + _dot_t(mean_b, w1lb_r[...])
        + b1_r[...] + _dot_t(x_r[...], w1r_r[...]))
  h1 = _elu(h1)
  p2_o[...] = _dot_t(h1, w2l_r[...])
  r2_o[...] = _dot_t(h1, w2r_r[...]) + b2_r[...]
  inv_o[...] = jnp.broadcast_to(inv, inv_o.shape)


def _t1(x, agg1a, agg1b, w1l, b1, w1r, w2l, b2, w2r):
  grid = (_N // _BM,)
  full = lambda shape: pl.BlockSpec(shape, lambda i: (0,) * len(shape))
  return pl.pallas_call(
      _t1_body,
      grid=grid,
      in_specs=[
          pl.BlockSpec((_BM, 128), lambda i: (i, 0)),
          pl.BlockSpec((_NC, _BM, 72), lambda i: (0, i, 0)),
          pl.BlockSpec((_NC, _BM, 64), lambda i: (0, i, 0)),
          full((256, 64)), full((256, 64)), full((1, 256)), full((256, 128)),
          full((32, 256)), full((1, 32)), full((32, 256)),
      ],
      out_specs=[
          pl.BlockSpec((_BM, 32), lambda i: (i, 0)),
          pl.BlockSpec((_BM, 32), lambda i: (i, 0)),
          pl.BlockSpec((_BM, 8), lambda i: (i, 0)),
      ],
      out_shape=[
          jax.ShapeDtypeStruct((_N, 32), jnp.float32),
          jax.ShapeDtypeStruct((_N, 32), jnp.float32),
          jax.ShapeDtypeStruct((_N, 8), jnp.float32),
      ],
  )(x, agg1a, agg1b, w1l[:, :64], w1l[:, 64:], b1, w1r,
    w2l, b2, w2r)


def _t2_body(a_r, i_r, r2_r, h2_o):
  mean = (a_r[0] + a_r[1]) * i_r[:, 0:1]
  h2_o[...] = _elu(mean + r2_r[...])


def _t2(agg2, inv, r2):
  grid = (_N // _BM,)
  return pl.pallas_call(
      _t2_body,
      grid=grid,
      in_specs=[
          pl.BlockSpec((_NC, _BM, 32), lambda i: (0, i, 0)),
          pl.BlockSpec((_BM, 8), lambda i: (i, 0)),
          pl.BlockSpec((_BM, 32), lambda i: (i, 0)),
      ],
      out_specs=pl.BlockSpec((_BM, 32), lambda i: (i, 0)),
      out_shape=jax.ShapeDtypeStruct((_N, 32), jnp.float32),
  )(agg2, inv, r2)


def _t3_body(a_r, i_r, h2_r, w3l_r, b3_r, w3r_r, out_o):
  mean = (a_r[0] + a_r[1]) * i_r[:, 0:1]
  out_o[...] = (_dot_t(mean, w3l_r[...]) + b3_r[...]
                + _dot_t(h2_r[...], w3r_r[...]))


def _t3(agg3, inv, h2, w3l, b3, w3r):
  grid = (_N // _BM,)
  full = lambda shape: pl.BlockSpec(shape, lambda i: (0,) * len(shape))
  return pl.pallas_call(
      _t3_body,
      grid=grid,
      in_specs=[
          pl.BlockSpec((_NC, _BM, 32), lambda i: (0, i, 0)),
          pl.BlockSpec((_BM, 8), lambda i: (i, 0)),
          pl.BlockSpec((_BM, 32), lambda i: (i, 0)),
          full((64, 32)), full((1, 64)), full((64, 32)),
      ],
      out_specs=pl.BlockSpec((_BM, 64), lambda i: (i, 0)),
      out_shape=jax.ShapeDtypeStruct((_N, 64), jnp.float32),
  )(agg3, inv, h2, w3l, b3, w3r)


def kernel(x, edge_index, W1l, b1, W1r, W2l, b2, W2r, W3l, b3, W3r):
  ei = edge_index.astype(jnp.int32)
  src = jnp.pad(ei[0], (0, _EPAD - _E))
  dst = jnp.pad(ei[1], (0, _EPAD - _E), constant_values=_PAD_NODE)
  idx3 = jnp.stack([src.reshape(_NW, _NCH, _C),
                    dst.reshape(_NW, _NCH, _C)], axis=2)

  xa = jnp.concatenate([x[:, :64], jnp.ones((_N, 8), jnp.float32)], axis=1)

  agg1a = _seg_sum(xa, idx3, 0, 72, False)[0]
  agg1b = _seg_sum(x, idx3, 64, 64, False)[0]
  p2, r2, inv = _t1(x, agg1a, agg1b, W1l, b1.reshape(1, 256), W1r,
                    W2l, b2.reshape(1, 32), W2r)
  agg2 = _seg_sum(p2, idx3, 0, 32, False)[0]
  h2 = _t2(agg2, inv, r2)
  agg3 = _seg_sum(h2, idx3, 0, 32, False)[0]
  return _t3(agg3, inv, h2, W3l, b3.reshape(1, 64), W3r)
